# Initial kernel scaffold; baseline (speedup 1.0000x reference)
import functools

import jax
import jax.numpy as jnp
from jax import lax
from jax.experimental import pallas as pl
from jax.experimental.pallas import tpu as pltpu
from jax.experimental.pallas import tpu_sc as plsc

N_NODES = 10000
N_EDGES = 320000
D_FEAT = 128
H1 = 64
H2 = 32
N_CLASSES = 41

NC = 2
NS = 16
NW = NC * NS
CH = 128
NCHUNK = -(-N_EDGES // (NW * CH))
EPAD = NW * NCHUNK * CH
NPAD = 10240
RPS = NPAD // NS
ZB = 16

_mesh = plsc.VectorSubcoreMesh(core_axis_name="c", subcore_axis_name="s")


@functools.partial(
    pl.kernel,
    out_type=jax.ShapeDtypeStruct((NC, NPAD), jnp.float32),
    mesh=_mesh,
    scratch_types=[
        pltpu.VMEM((NCHUNK, CH), jnp.int32),
        pltpu.VMEM((CH,), jnp.float32),
        pltpu.VMEM((RPS,), jnp.float32),
        pltpu.VMEM_SHARED((NPAD,), jnp.float32),
    ],
)
def _deg_kernel(dst_hbm, out_hbm, dst_v, ones_v, zero_v, acc_sh):
    cid = lax.axis_index("c")
    sid = lax.axis_index("s")
    wid = sid * NC + cid
    for i in range(CH // 16):
        ones_v[pl.ds(i * 16, 16)] = jnp.ones((16,), jnp.float32)
    for i in range(RPS // 16):
        zero_v[pl.ds(i * 16, 16)] = jnp.zeros((16,), jnp.float32)
    pltpu.sync_copy(zero_v, acc_sh.at[pl.ds(sid * RPS, RPS)])
    plsc.subcore_barrier()
    pltpu.sync_copy(dst_hbm.at[wid], dst_v)

    def body(j, carry):
        pltpu.sync_copy(ones_v, acc_sh.at[dst_v.at[j]], add=True)
        return carry

    lax.fori_loop(0, NCHUNK, body, 0)
    plsc.subcore_barrier()
    pltpu.sync_copy(acc_sh.at[pl.ds(sid * RPS, RPS)],
                    out_hbm.at[cid, pl.ds(sid * RPS, RPS)])


def _make_agg(D):
    @functools.partial(
        pl.kernel,
        out_type=jax.ShapeDtypeStruct((NC, NPAD, D), jnp.float32),
        mesh=_mesh,
        scratch_types=[
            pltpu.VMEM((NCHUNK, CH), jnp.int32),
            pltpu.VMEM((NCHUNK, CH), jnp.int32),
            pltpu.VMEM((CH, D), jnp.float32),
            pltpu.VMEM((ZB, D), jnp.float32),
            pltpu.VMEM_SHARED((NPAD, D), jnp.float32),
            pltpu.SemaphoreType.DMA,
        ],
    )
    def agg(g_hbm, src_hbm, dst_hbm, out_hbm, src_v, dst_v, rows_v, zb_v,
            acc_sh, sem):
        cid = lax.axis_index("c")
        sid = lax.axis_index("s")
        wid = sid * NC + cid
        for i in range(ZB):
            for j in range(D // 16):
                zb_v[i, pl.ds(j * 16, 16)] = jnp.zeros((16,), jnp.float32)

        def zbody(t, carry):
            pltpu.sync_copy(zb_v, acc_sh.at[pl.ds(sid * RPS + t * ZB, ZB)])
            return carry

        lax.fori_loop(0, RPS // ZB, zbody, 0)
        plsc.subcore_barrier()
        pltpu.sync_copy(src_hbm.at[wid], src_v)
        pltpu.sync_copy(dst_hbm.at[wid], dst_v)

        def body(j, carry):
            pltpu.async_copy(g_hbm.at[src_v.at[j]], rows_v, sem).wait()
            pltpu.sync_copy(rows_v, acc_sh.at[dst_v.at[j]], add=True)
            return carry

        lax.fori_loop(0, NCHUNK, body, 0)
        plsc.subcore_barrier()
        pltpu.sync_copy(acc_sh.at[pl.ds(sid * RPS, RPS)],
                        out_hbm.at[cid, pl.ds(sid * RPS, RPS)])

    return agg


_agg64 = _make_agg(H1)
_agg32 = _make_agg(H2)


def _tc1_body(x_ref, w_ref, b_ref, degp_ref, g_ref, dinv_ref):
    deg = degp_ref[0] + degp_ref[1] + 1.0
    dinv = lax.rsqrt(deg)
    h = jnp.dot(x_ref[...], w_ref[...],
                preferred_element_type=jnp.float32) + b_ref[...]
    g_ref[...] = dinv * h
    dinv_ref[...] = dinv


def _tc2_body(aggp_ref, g1_ref, dinv_ref, w_ref, b_ref, g2_ref):
    dinv = dinv_ref[...]
    h1 = jnp.maximum(dinv * (aggp_ref[0] + aggp_ref[1] + g1_ref[...]), 0.0)
    g2_ref[...] = dinv * (
        jnp.dot(h1, w_ref[...], preferred_element_type=jnp.float32)
        + b_ref[...])


def _tc3_body(aggp_ref, g2_ref, dinv_ref, wf_ref, bf_ref, out_ref):
    dinv = dinv_ref[...]
    h2 = jnp.maximum(dinv * (aggp_ref[0] + aggp_ref[1] + g2_ref[...]), 0.0)
    row = lax.broadcasted_iota(jnp.int32, (NPAD, 1), 0)
    h2 = jnp.where(row < N_NODES, h2, 0.0)
    m = jnp.sum(h2, axis=0, keepdims=True) * (1.0 / N_NODES)
    out_ref[...] = jnp.dot(m, wf_ref[...],
                           preferred_element_type=jnp.float32) + bf_ref[...]


def kernel(x, edge_index, W1, b1, W2, b2, Wf, bf):
    src = edge_index[0].astype(jnp.int32)
    dst = edge_index[1].astype(jnp.int32)
    pad = EPAD - N_EDGES
    srcp = jnp.concatenate(
        [src, jnp.full((pad,), N_NODES, jnp.int32)]).reshape(NW, NCHUNK, CH)
    dstp = jnp.concatenate(
        [dst, jnp.full((pad,), N_NODES, jnp.int32)]).reshape(NW, NCHUNK, CH)
    xp = jnp.pad(x, ((0, NPAD - N_NODES), (0, 0)))

    degp = _deg_kernel(dstp)
    degp3 = degp.reshape(NC, NPAD, 1)

    g1, dinv = pl.pallas_call(
        _tc1_body,
        out_shape=[
            jax.ShapeDtypeStruct((NPAD, H1), jnp.float32),
            jax.ShapeDtypeStruct((NPAD, 1), jnp.float32),
        ],
    )(xp, W1, b1.reshape(1, H1), degp3)

    aggp1 = _agg64(g1, srcp, dstp)

    g2 = pl.pallas_call(
        _tc2_body,
        out_shape=jax.ShapeDtypeStruct((NPAD, H2), jnp.float32),
    )(aggp1, g1, dinv, W2, b2.reshape(1, H2))

    aggp2 = _agg32(g2, srcp, dstp)

    out = pl.pallas_call(
        _tc3_body,
        out_shape=jax.ShapeDtypeStruct((1, N_CLASSES), jnp.float32),
    )(aggp2, g2, dinv, Wf, bf.reshape(1, N_CLASSES))

    return out.reshape(N_CLASSES)


# speedup vs baseline: 23.7452x; 23.7452x over previous
import functools

import jax
import jax.numpy as jnp
from jax import lax
from jax.experimental import pallas as pl
from jax.experimental.pallas import tpu as pltpu
from jax.experimental.pallas import tpu_sc as plsc

N_NODES = 10000
N_EDGES = 320000
D_FEAT = 128
H1 = 64
H2 = 32
N_CLASSES = 41

NC = 2
NS = 16
NW = NC * NS
CH = 128
NCHUNK = -(-N_EDGES // (NW * CH))
EPAD = NW * NCHUNK * CH
NPAD = 10240
RPS = NPAD // NS
ZB = 16

_mesh = plsc.VectorSubcoreMesh(core_axis_name="c", subcore_axis_name="s")


@functools.partial(
    pl.kernel,
    out_type=jax.ShapeDtypeStruct((NC, NPAD), jnp.float32),
    mesh=_mesh,
    scratch_types=[
        pltpu.VMEM((NCHUNK, CH), jnp.int32),
        pltpu.VMEM((CH,), jnp.float32),
        pltpu.VMEM((RPS,), jnp.float32),
        pltpu.VMEM_SHARED((NPAD,), jnp.float32),
    ],
    compiler_params=pltpu.CompilerParams(use_tc_tiling_on_sc=False),
)
def _deg_kernel(dst_hbm, out_hbm, dst_v, ones_v, zero_v, acc_sh):
    cid = lax.axis_index("c")
    sid = lax.axis_index("s")
    wid = sid * NC + cid
    for i in range(CH // 16):
        ones_v[pl.ds(i * 16, 16)] = jnp.ones((16,), jnp.float32)
    for i in range(RPS // 16):
        zero_v[pl.ds(i * 16, 16)] = jnp.zeros((16,), jnp.float32)
    pltpu.sync_copy(zero_v, acc_sh.at[pl.ds(sid * RPS, RPS)])
    plsc.subcore_barrier()
    pltpu.sync_copy(dst_hbm.at[wid], dst_v)

    def body(j, carry):
        pltpu.sync_copy(ones_v, acc_sh.at[dst_v.at[j]], add=True)
        return carry

    lax.fori_loop(0, NCHUNK, body, 0)
    plsc.subcore_barrier()
    pltpu.sync_copy(acc_sh.at[pl.ds(sid * RPS, RPS)],
                    out_hbm.at[cid, pl.ds(sid * RPS, RPS)])


def _make_agg(D):
    @functools.partial(
        pl.kernel,
        out_type=jax.ShapeDtypeStruct((NC, NPAD, D), jnp.float32),
        mesh=_mesh,
        scratch_types=[
            pltpu.VMEM((NCHUNK, CH), jnp.int32),
            pltpu.VMEM((NCHUNK, CH), jnp.int32),
            pltpu.VMEM((CH, D), jnp.float32),
            pltpu.VMEM((ZB, D), jnp.float32),
            pltpu.VMEM_SHARED((NPAD, D), jnp.float32),
            pltpu.SemaphoreType.DMA,
        ],
        compiler_params=pltpu.CompilerParams(use_tc_tiling_on_sc=False),
    )
    def agg(g_hbm, src_hbm, dst_hbm, out_hbm, src_v, dst_v, rows_v, zb_v,
            acc_sh, sem):
        cid = lax.axis_index("c")
        sid = lax.axis_index("s")
        wid = sid * NC + cid
        for i in range(ZB):
            for j in range(D // 16):
                zb_v[i, pl.ds(j * 16, 16)] = jnp.zeros((16,), jnp.float32)

        def zbody(t, carry):
            pltpu.sync_copy(zb_v, acc_sh.at[pl.ds(sid * RPS + t * ZB, ZB)])
            return carry

        lax.fori_loop(0, RPS // ZB, zbody, 0)
        plsc.subcore_barrier()
        pltpu.sync_copy(src_hbm.at[wid], src_v)
        pltpu.sync_copy(dst_hbm.at[wid], dst_v)

        def body(j, carry):
            pltpu.async_copy(g_hbm.at[src_v.at[j]], rows_v, sem).wait()
            pltpu.sync_copy(rows_v, acc_sh.at[dst_v.at[j]], add=True)
            return carry

        lax.fori_loop(0, NCHUNK, body, 0)
        plsc.subcore_barrier()
        pltpu.sync_copy(acc_sh.at[pl.ds(sid * RPS, RPS)],
                        out_hbm.at[cid, pl.ds(sid * RPS, RPS)])

    return agg


_agg64 = _make_agg(H1)
_agg32 = _make_agg(H2)


def _tc1_body(x_ref, w_ref, b_ref, degp_ref, g_ref, dinv_ref):
    deg = degp_ref[0] + degp_ref[1] + 1.0
    dinv = lax.rsqrt(deg)
    h = jnp.dot(x_ref[...], w_ref[...],
                preferred_element_type=jnp.float32) + b_ref[...]
    g_ref[...] = dinv * h
    dinv_ref[...] = dinv


def _tc2_body(aggp_ref, g1_ref, dinv_ref, w_ref, b_ref, g2_ref):
    dinv = dinv_ref[...]
    h1 = jnp.maximum(dinv * (aggp_ref[0] + aggp_ref[1] + g1_ref[...]), 0.0)
    g2_ref[...] = dinv * (
        jnp.dot(h1, w_ref[...], preferred_element_type=jnp.float32)
        + b_ref[...])


def _tc3_body(aggp_ref, g2_ref, dinv_ref, wf_ref, bf_ref, out_ref):
    dinv = dinv_ref[...]
    h2 = jnp.maximum(dinv * (aggp_ref[0] + aggp_ref[1] + g2_ref[...]), 0.0)
    row = lax.broadcasted_iota(jnp.int32, (NPAD, 1), 0)
    h2 = jnp.where(row < N_NODES, h2, 0.0)
    m = jnp.sum(h2, axis=0, keepdims=True) * (1.0 / N_NODES)
    out_ref[...] = jnp.dot(m, wf_ref[...],
                           preferred_element_type=jnp.float32) + bf_ref[...]


def kernel(x, edge_index, W1, b1, W2, b2, Wf, bf):
    src = edge_index[0].astype(jnp.int32)
    dst = edge_index[1].astype(jnp.int32)
    pad = EPAD - N_EDGES
    srcp = jnp.concatenate(
        [src, jnp.full((pad,), N_NODES, jnp.int32)]).reshape(NW, NCHUNK, CH)
    dstp = jnp.concatenate(
        [dst, jnp.full((pad,), N_NODES, jnp.int32)]).reshape(NW, NCHUNK, CH)
    xp = jnp.pad(x, ((0, NPAD - N_NODES), (0, 0)))

    degp = _deg_kernel(dstp)
    degp3 = degp.reshape(NC, NPAD, 1)

    g1, dinv = pl.pallas_call(
        _tc1_body,
        out_shape=[
            jax.ShapeDtypeStruct((NPAD, H1), jnp.float32),
            jax.ShapeDtypeStruct((NPAD, 1), jnp.float32),
        ],
    )(xp, W1, b1.reshape(1, H1), degp3)

    aggp1 = _agg64(g1, srcp, dstp)

    g2 = pl.pallas_call(
        _tc2_body,
        out_shape=jax.ShapeDtypeStruct((NPAD, H2), jnp.float32),
    )(aggp1, g1, dinv, W2, b2.reshape(1, H2))

    aggp2 = _agg32(g2, srcp, dstp)

    out = pl.pallas_call(
        _tc3_body,
        out_shape=jax.ShapeDtypeStruct((1, N_CLASSES), jnp.float32),
    )(aggp2, g2, dinv, Wf, bf.reshape(1, N_CLASSES))

    return out.reshape(N_CLASSES)
